# Initial kernel scaffold; baseline (speedup 1.0000x reference)
import jax
import jax.numpy as jnp
from jax.experimental import pallas as pl


def kernel(x, lengths, cap_obj_nums, cap_pred_nums, cap_obj_list, cap_rel_list, word_embed, W_ih, W_hh, b_ih, b_hh, obj_embed, rel_embed, lin_W, lin_b):
    raise NotImplementedError("write your pallas kernel here")


# speedup vs baseline: 1.3295x; 1.3295x over previous
import functools

import jax
import jax.numpy as jnp
from jax import lax
from jax.experimental import pallas as pl
from jax.experimental.pallas import tpu as pltpu
from jax.experimental.pallas import tpu_sc as plsc

EMBED = 1024
CHUNK = 80
PCHUNK = 40


def _copy_body(idx_ref, src_ref, out_ref):
    del idx_ref
    out_ref[...] = src_ref[...]


def _gather_rows_tc(table, idx):
    n = idx.shape[0]
    v, d = table.shape
    table3 = table.reshape(v, 1, d)
    grid_spec = pltpu.PrefetchScalarGridSpec(
        num_scalar_prefetch=1,
        grid=(n,),
        in_specs=[pl.BlockSpec((1, 1, d), lambda i, idx_ref: (idx_ref[i], 0, 0))],
        out_specs=pl.BlockSpec((1, 1, d), lambda i, idx_ref: (i, 0, 0)),
    )
    out = pl.pallas_call(
        _copy_body,
        grid_spec=grid_spec,
        out_shape=jax.ShapeDtypeStruct((n, 1, d), table.dtype),
    )(idx, table3)
    return out.reshape(n, d)


def _dense_body(xe_ref, wih_ref, bih_ref, bhh_ref, len_ref,
                obj_ref, linw_ref, linb_ref, cap_ref, proj_ref):
    gi = lax.dot_general(xe_ref[...], wih_ref[...], (((1,), (1,)), ((), ())),
                         preferred_element_type=jnp.float32) + bih_ref[...]
    bhh = bhh_ref[...]
    i_r = gi[:, :EMBED]
    i_z = gi[:, EMBED:2 * EMBED]
    i_n = gi[:, 2 * EMBED:]
    h_r = bhh[:, :EMBED]
    h_z = bhh[:, EMBED:2 * EMBED]
    h_n = bhh[:, 2 * EMBED:]
    r = jax.nn.sigmoid(i_r + h_r)
    z = jax.nn.sigmoid(i_z + h_z)
    n = jnp.tanh(i_n + r * h_n)
    h_new = (1.0 - z) * n
    mask = 0 < len_ref[...]
    out = jnp.where(mask, h_new, 0.0)
    norm = jnp.sqrt(jnp.sum(out * out, axis=1, keepdims=True)) + 1e-8
    cap_ref[...] = out / norm
    proj_ref[...] = lax.dot_general(obj_ref[...], linw_ref[...],
                                    (((1,), (1,)), ((), ())),
                                    preferred_element_type=jnp.float32) \
        + linb_ref[...]


def _dense_tc(xe0, W_ih, b_ih, b_hh, lengths, obj_embed, lin_W, lin_b):
    b = xe0.shape[0]
    nobj = obj_embed.shape[0]
    gconv = lin_W.shape[0]
    return pl.pallas_call(
        _dense_body,
        out_shape=(jax.ShapeDtypeStruct((b, EMBED), jnp.float32),
                   jax.ShapeDtypeStruct((nobj, gconv), jnp.float32)),
    )(xe0, W_ih, b_ih.reshape(1, -1), b_hh.reshape(1, -1),
      lengths.reshape(b, 1), obj_embed, lin_W, lin_b.reshape(1, -1))


def _sc_gathers(p2_idx, obj_idx, pair_table, proj, dp):
    e2 = p2_idx.shape[0]
    o = obj_idx.shape[0]
    dpair = 2 * dp
    dpad = pair_table.shape[1]
    do = proj.shape[1]
    info = plsc.get_sparse_core_info()
    nw = info.num_cores * info.num_subcores
    n_pred_chunks = e2 // PCHUNK
    n_obj_chunks = o // CHUNK
    pred_iters = -(-n_pred_chunks // nw)
    obj_iters = -(-n_obj_chunks // nw)
    mesh = plsc.VectorSubcoreMesh(core_axis_name="c", subcore_axis_name="s")

    @functools.partial(
        pl.kernel, mesh=mesh,
        compiler_params=pltpu.CompilerParams(use_tc_tiling_on_sc=False),
        out_type=(jax.ShapeDtypeStruct((e2, dpair), jnp.float32),
                  jax.ShapeDtypeStruct((o, do), jnp.float32)),
        scratch_types=[
            pltpu.VMEM((PCHUNK,), jnp.int32),
            pltpu.VMEM((CHUNK,), jnp.int32),
            pltpu.VMEM((PCHUNK, dpad), jnp.float32),
            pltpu.VMEM((CHUNK, do), jnp.float32),
            pltpu.SemaphoreType.DMA,
        ],
    )
    def k(pidx_hbm, oidx_hbm, pair_hbm, proj_hbm, pred_out, obj_out,
          pidx_v, oidx_v, prow_v, orow_v, sem):
        wid = lax.axis_index("s") * info.num_cores + lax.axis_index("c")

        def pred_body(it, carry):
            chunk = it * nw + wid

            @pl.when(chunk < n_pred_chunks)
            def _():
                base = chunk * PCHUNK
                pltpu.sync_copy(pidx_hbm.at[pl.ds(base, PCHUNK)], pidx_v)
                pltpu.async_copy(pair_hbm.at[pidx_v], prow_v, sem).wait()
                pltpu.sync_copy(prow_v.at[:, pl.ds(0, dpair)],
                                pred_out.at[pl.ds(base, PCHUNK)])
            return carry

        lax.fori_loop(0, pred_iters, pred_body, 0)

        def obj_body(it, carry):
            chunk = it * nw + wid

            @pl.when(chunk < n_obj_chunks)
            def _():
                base = chunk * CHUNK
                pltpu.sync_copy(oidx_hbm.at[pl.ds(base, CHUNK)], oidx_v)
                pltpu.async_copy(proj_hbm.at[oidx_v], orow_v, sem).wait()
                pltpu.sync_copy(orow_v, obj_out.at[pl.ds(base, CHUNK)])
            return carry

        lax.fori_loop(0, obj_iters, obj_body, 0)

    return k(p2_idx, obj_idx, pair_table, proj)


def kernel(x, lengths, cap_obj_nums, cap_pred_nums, cap_obj_list, cap_rel_list,
           word_embed, W_ih, W_hh, b_ih, b_hh, obj_embed, rel_embed,
           lin_W, lin_b):
    del cap_obj_nums, cap_pred_nums, W_hh
    b = x.shape[0]
    e = cap_rel_list.shape[0]
    v, dp = rel_embed.shape
    x0 = x[:, 0]
    p_idx = cap_rel_list[:, 1]
    pair_table = jnp.concatenate(
        [jnp.broadcast_to(rel_embed[:, None, :], (v, v, dp)),
         jnp.broadcast_to(rel_embed[None, :, :], (v, v, dp)),
         jnp.zeros((v, v, 8), jnp.float32)],
        axis=2).reshape(v * v, 2 * dp + 8)
    p2_idx = p_idx[0::2] * v + p_idx[1::2]
    xe0 = _gather_rows_tc(word_embed, x0)
    cap, proj = _dense_tc(xe0, W_ih, b_ih, b_hh, lengths,
                          obj_embed, lin_W, lin_b)
    pred2, obj_vecs = _sc_gathers(p2_idx, cap_obj_list, pair_table, proj, dp)
    pred_vecs = pred2.reshape(e, dp)
    cap_emb = cap.reshape(b, 1, EMBED)
    return (cap_emb, lengths, obj_vecs, pred_vecs)
